# baseline (speedup 1.0000x reference)
import functools

import jax
import jax.numpy as jnp
from jax import lax
from jax.experimental import pallas as pl
from jax.experimental.pallas import tpu as pltpu
from jax.experimental.pallas import tpu_sc as plsc

_CAPACITY = 1000000
_MEMORY_DIM = 32
_B = 4096
_LBYTES = 200
_NOUT = 198


def _hash16(b0, b1, b2):
    t = (1255593 + b0) * 961 + b1 * 31 + b2
    t = t - (t >> 20) * _CAPACITY
    t = t - (t >> 20) * _CAPACITY
    t = t - (t >> 20) * _CAPACITY
    return jnp.where(t >= _CAPACITY, t - _CAPACITY, t)


def _sc_kernel(bytes_hbm, table_hbm, out_hbm,
               bytes_v, idx_v, rows0, rows1, sg0, sg1, so0, so1):
    info = plsc.get_sparse_core_info()
    nc = info.num_cores
    wid = lax.axis_index("s") * nc + lax.axis_index("c")
    nw = nc * info.num_subcores
    rows_per_w = _B // nw

    nbytes = rows_per_w * _LBYTES
    pltpu.sync_copy(bytes_hbm.at[pl.ds(wid * nbytes, nbytes)],
                    bytes_v.at[pl.ds(0, nbytes)])

    def hash_row(rr, _):
        bbase = rr * _LBYTES
        for c in range(13):
            b0 = bytes_v[pl.ds(bbase + c * 16, 16)]
            b1 = bytes_v[pl.ds(bbase + c * 16 + 1, 16)]
            b2 = bytes_v[pl.ds(bbase + c * 16 + 2, 16)]
            idx_v[pl.ds(bbase + c * 16, 16)] = _hash16(b0, b1, b2)
        return 0

    lax.fori_loop(0, rows_per_w, hash_row, 0)

    wbase = wid * rows_per_w

    def drain(rows_v, sem):
        pltpu.make_async_copy(rows_v.at[:, pl.ds(0, _MEMORY_DIM)],
                              out_hbm.at[wbase], sem).wait()

    def gather(rr, rows_v, sem):
        ib = rr * _LBYTES
        c0 = pltpu.async_copy(table_hbm.at[idx_v.at[pl.ds(ib, 128)]],
                              rows_v.at[pl.ds(0, 128)], sem)
        c1 = pltpu.async_copy(table_hbm.at[idx_v.at[pl.ds(ib + 128, 70)]],
                              rows_v.at[pl.ds(128, 70)], sem)
        return c0, c1

    def pair(i, _):
        @pl.when(i > 0)
        def _():
            drain(rows0, so0)
            drain(rows1, so1)
        g0a, g0b = gather(2 * i, rows0, sg0)
        g1a, g1b = gather(2 * i + 1, rows1, sg1)
        g0a.wait()
        g0b.wait()
        pltpu.async_copy(rows0.at[:, pl.ds(0, _MEMORY_DIM)],
                         out_hbm.at[wbase + 2 * i], so0)
        g1a.wait()
        g1b.wait()
        pltpu.async_copy(rows1.at[:, pl.ds(0, _MEMORY_DIM)],
                         out_hbm.at[wbase + 2 * i + 1], so1)
        return 0

    lax.fori_loop(0, rows_per_w // 2, pair, 0)
    drain(rows0, so0)
    drain(rows1, so1)


def kernel(input_bytes, memory_table):
    mesh = plsc.VectorSubcoreMesh(core_axis_name="c", subcore_axis_name="s")
    k = functools.partial(
        pl.kernel,
        mesh=mesh,
        compiler_params=pltpu.CompilerParams(use_tc_tiling_on_sc=False),
        out_type=jax.ShapeDtypeStruct((_B, _NOUT, _MEMORY_DIM), jnp.float32),
        scratch_types=[
            pltpu.VMEM((_B // 32 * _LBYTES + 16,), jnp.int32),
            pltpu.VMEM((_B // 32 * _LBYTES + 16,), jnp.int32),
            pltpu.VMEM((_NOUT, 128), jnp.float32),
            pltpu.VMEM((_NOUT, 128), jnp.float32),
            pltpu.SemaphoreType.DMA,
            pltpu.SemaphoreType.DMA,
            pltpu.SemaphoreType.DMA,
            pltpu.SemaphoreType.DMA,
        ],
    )(_sc_kernel)
    table_pad = jnp.pad(memory_table, ((0, 0), (0, 128 - _MEMORY_DIM)))
    return k(input_bytes.reshape(-1), table_pad)


# speedup vs baseline: 1.2063x; 1.2063x over previous
import functools

import jax
import jax.numpy as jnp
from jax import lax
from jax.experimental import pallas as pl
from jax.experimental.pallas import tpu as pltpu
from jax.experimental.pallas import tpu_sc as plsc

_CAPACITY = 1000000
_MEMORY_DIM = 32
_B = 4096
_LBYTES = 200
_NOUT = 198


def _hash16(b0, b1, b2):
    t = (1255593 + b0) * 961 + b1 * 31 + b2
    t = t - (t >> 20) * _CAPACITY
    t = t - (t >> 20) * _CAPACITY
    t = t - (t >> 20) * _CAPACITY
    return jnp.where(t >= _CAPACITY, t - _CAPACITY, t)


def _sc_kernel(bytes_hbm, table_hbm, out_hbm,
               bytes_v, idx_v, rows0, rows1, sg0, sg1, so0, so1):
    info = plsc.get_sparse_core_info()
    nc = info.num_cores
    wid = lax.axis_index("s") * nc + lax.axis_index("c")
    nw = nc * info.num_subcores
    rows_per_w = _B // nw

    nbytes = rows_per_w * _LBYTES
    pltpu.sync_copy(bytes_hbm.at[pl.ds(wid * nbytes, nbytes)],
                    bytes_v.at[pl.ds(0, nbytes)])

    def hash_row(rr, _):
        bbase = rr * _LBYTES
        for c in range(13):
            b0 = bytes_v[pl.ds(bbase + c * 16, 16)]
            b1 = bytes_v[pl.ds(bbase + c * 16 + 1, 16)]
            b2 = bytes_v[pl.ds(bbase + c * 16 + 2, 16)]
            idx_v[pl.ds(bbase + c * 16, 16)] = _hash16(b0, b1, b2)
        return 0

    lax.fori_loop(0, rows_per_w, hash_row, 0)

    wbase = wid * rows_per_w

    def drain(rows_v, sem):
        pltpu.make_async_copy(rows_v, out_hbm.at[wbase], sem).wait()

    def gather(rr, rows_v, sem):
        ib = rr * _LBYTES
        c0 = pltpu.async_copy(table_hbm.at[idx_v.at[pl.ds(ib, 128)]],
                              rows_v.at[pl.ds(0, 128)], sem)
        c1 = pltpu.async_copy(table_hbm.at[idx_v.at[pl.ds(ib + 128, 70)]],
                              rows_v.at[pl.ds(128, 70)], sem)
        return c0, c1

    def pair(i, _):
        @pl.when(i > 0)
        def _():
            drain(rows0, so0)
            drain(rows1, so1)
        g0a, g0b = gather(2 * i, rows0, sg0)
        g1a, g1b = gather(2 * i + 1, rows1, sg1)
        g0a.wait()
        g0b.wait()
        pltpu.async_copy(rows0, out_hbm.at[wbase + 2 * i], so0)
        g1a.wait()
        g1b.wait()
        pltpu.async_copy(rows1, out_hbm.at[wbase + 2 * i + 1], so1)
        return 0

    lax.fori_loop(0, rows_per_w // 2, pair, 0)
    drain(rows0, so0)
    drain(rows1, so1)


def kernel(input_bytes, memory_table):
    mesh = plsc.VectorSubcoreMesh(core_axis_name="c", subcore_axis_name="s")
    k = functools.partial(
        pl.kernel,
        mesh=mesh,
        compiler_params=pltpu.CompilerParams(use_tc_tiling_on_sc=False),
        out_type=jax.ShapeDtypeStruct((_B, _NOUT, _MEMORY_DIM), jnp.float32),
        scratch_types=[
            pltpu.VMEM((_B // 32 * _LBYTES + 16,), jnp.int32),
            pltpu.VMEM((_B // 32 * _LBYTES + 16,), jnp.int32),
            pltpu.VMEM((_NOUT, _MEMORY_DIM), jnp.float32),
            pltpu.VMEM((_NOUT, _MEMORY_DIM), jnp.float32),
            pltpu.SemaphoreType.DMA,
            pltpu.SemaphoreType.DMA,
            pltpu.SemaphoreType.DMA,
            pltpu.SemaphoreType.DMA,
        ],
    )(_sc_kernel)
    table_lin = _tc_pack_table(memory_table.T).reshape(_CAPACITY, _MEMORY_DIM)
    return k(input_bytes.reshape(-1), table_lin)


_TCOLS = 2048


def _tc_pack_body(x_ref, o_ref):
    y3 = x_ref[...].T.reshape(_TCOLS // 4, 4, _MEMORY_DIM)
    for j in range(4):
        o_ref[:, 32 * j:32 * (j + 1)] = y3[:, j, :]


def _tc_pack_table(table_t):
    return pl.pallas_call(
        _tc_pack_body,
        grid=((_CAPACITY + _TCOLS - 1) // _TCOLS,),
        in_specs=[pl.BlockSpec((_MEMORY_DIM, _TCOLS), lambda i: (0, i))],
        out_specs=pl.BlockSpec((_TCOLS // 4, 128), lambda i: (i, 0)),
        out_shape=jax.ShapeDtypeStruct((_CAPACITY // 4, 128), jnp.float32),
    )(table_t)


# speedup vs baseline: 1.2100x; 1.0031x over previous
import functools

import jax
import jax.numpy as jnp
from jax import lax
from jax.experimental import pallas as pl
from jax.experimental.pallas import tpu as pltpu
from jax.experimental.pallas import tpu_sc as plsc

_CAPACITY = 1000000
_MEMORY_DIM = 32
_B = 4096
_LBYTES = 200
_NOUT = 198


def _hash16(b0, b1, b2):
    t = (1255593 + b0) * 961 + b1 * 31 + b2
    t = t - (t >> 20) * _CAPACITY
    t = t - (t >> 20) * _CAPACITY
    t = t - (t >> 20) * _CAPACITY
    return jnp.where(t >= _CAPACITY, t - _CAPACITY, t)


def _sc_hash_kernel(bytes_hbm, idx_hbm, bytes_v, idx_v):
    info = plsc.get_sparse_core_info()
    nc = info.num_cores
    wid = lax.axis_index("s") * nc + lax.axis_index("c")
    nw = nc * info.num_subcores
    rows_per_w = _B // nw

    nbytes = rows_per_w * _LBYTES
    pltpu.sync_copy(bytes_hbm.at[pl.ds(wid * nbytes, nbytes)],
                    bytes_v.at[pl.ds(0, nbytes)])

    def hash_row(rr, _):
        bbase = rr * _LBYTES
        for c in range(13):
            b0 = bytes_v[pl.ds(bbase + c * 16, 16)]
            b1 = bytes_v[pl.ds(bbase + c * 16 + 1, 16)]
            b2 = bytes_v[pl.ds(bbase + c * 16 + 2, 16)]
            idx_v[pl.ds(bbase + c * 16, 16)] = _hash16(b0, b1, b2)
        return 0

    lax.fori_loop(0, rows_per_w, hash_row, 0)
    pltpu.sync_copy(idx_v.at[pl.ds(0, nbytes)],
                    idx_hbm.at[pl.ds(wid * nbytes, nbytes)])


def _sc_gather_kernel(idx_hbm, table_hbm, out_hbm,
                      idx_v, rows0, rows1, sg0, sg1, so0, so1):
    info = plsc.get_sparse_core_info()
    nc = info.num_cores
    wid = lax.axis_index("s") * nc + lax.axis_index("c")
    nw = nc * info.num_subcores
    rows_per_w = _B // nw

    nidx = rows_per_w * _LBYTES
    pltpu.sync_copy(idx_hbm.at[pl.ds(wid * nidx, nidx)],
                    idx_v.at[pl.ds(0, nidx)])

    wbase = wid * rows_per_w

    def drain(rows_v, sem):
        pltpu.make_async_copy(rows_v, out_hbm.at[wbase], sem).wait()

    def gather(rr, rows_v, sem):
        ib = rr * _LBYTES
        c0 = pltpu.async_copy(table_hbm.at[idx_v.at[pl.ds(ib, 128)]],
                              rows_v.at[pl.ds(0, 128)], sem)
        c1 = pltpu.async_copy(table_hbm.at[idx_v.at[pl.ds(ib + 128, 70)]],
                              rows_v.at[pl.ds(128, 70)], sem)
        return c0, c1

    def pair(i, _):
        @pl.when(i > 0)
        def _():
            drain(rows0, so0)
            drain(rows1, so1)
        g0a, g0b = gather(2 * i, rows0, sg0)
        g1a, g1b = gather(2 * i + 1, rows1, sg1)
        g0a.wait()
        g0b.wait()
        pltpu.async_copy(rows0, out_hbm.at[wbase + 2 * i], so0)
        g1a.wait()
        g1b.wait()
        pltpu.async_copy(rows1, out_hbm.at[wbase + 2 * i + 1], so1)
        return 0

    lax.fori_loop(0, rows_per_w // 2, pair, 0)
    drain(rows0, so0)
    drain(rows1, so1)


def kernel(input_bytes, memory_table):
    mesh = plsc.VectorSubcoreMesh(core_axis_name="c", subcore_axis_name="s")
    hash_k = functools.partial(
        pl.kernel,
        mesh=mesh,
        compiler_params=pltpu.CompilerParams(use_tc_tiling_on_sc=False),
        out_type=jax.ShapeDtypeStruct((_B * _LBYTES,), jnp.int32),
        scratch_types=[
            pltpu.VMEM((_B // 32 * _LBYTES + 16,), jnp.int32),
            pltpu.VMEM((_B // 32 * _LBYTES + 16,), jnp.int32),
        ],
    )(_sc_hash_kernel)
    gather_k = functools.partial(
        pl.kernel,
        mesh=mesh,
        compiler_params=pltpu.CompilerParams(use_tc_tiling_on_sc=False),
        out_type=jax.ShapeDtypeStruct((_B, _NOUT, _MEMORY_DIM), jnp.float32),
        scratch_types=[
            pltpu.VMEM((_B // 32 * _LBYTES + 16,), jnp.int32),
            pltpu.VMEM((_NOUT, _MEMORY_DIM), jnp.float32),
            pltpu.VMEM((_NOUT, _MEMORY_DIM), jnp.float32),
            pltpu.SemaphoreType.DMA,
            pltpu.SemaphoreType.DMA,
            pltpu.SemaphoreType.DMA,
            pltpu.SemaphoreType.DMA,
        ],
    )(_sc_gather_kernel)
    idx = hash_k(input_bytes.reshape(-1))
    table_lin = _tc_pack_table(memory_table.T).reshape(_CAPACITY, _MEMORY_DIM)
    return gather_k(idx, table_lin)


_TCOLS = 2048


def _tc_pack_body(x_ref, o_ref):
    y3 = x_ref[...].T.reshape(_TCOLS // 4, 4, _MEMORY_DIM)
    for j in range(4):
        o_ref[:, 32 * j:32 * (j + 1)] = y3[:, j, :]


def _tc_pack_table(table_t):
    return pl.pallas_call(
        _tc_pack_body,
        grid=((_CAPACITY + _TCOLS - 1) // _TCOLS,),
        in_specs=[pl.BlockSpec((_MEMORY_DIM, _TCOLS), lambda i: (0, i))],
        out_specs=pl.BlockSpec((_TCOLS // 4, 128), lambda i: (i, 0)),
        out_shape=jax.ShapeDtypeStruct((_CAPACITY // 4, 128), jnp.float32),
    )(table_t)


# speedup vs baseline: 1.3725x; 1.1343x over previous
import functools

import jax
import jax.numpy as jnp
from jax import lax
from jax.experimental import pallas as pl
from jax.experimental.pallas import tpu as pltpu
from jax.experimental.pallas import tpu_sc as plsc

_CAPACITY = 1000000
_MEMORY_DIM = 32
_B = 4096
_LBYTES = 200
_NOUT = 198


def _hash16(b0, b1, b2):
    t = (1255593 + b0) * 961 + b1 * 31 + b2
    t = t - (t >> 20) * _CAPACITY
    t = t - (t >> 20) * _CAPACITY
    t = t - (t >> 20) * _CAPACITY
    return jnp.where(t >= _CAPACITY, t - _CAPACITY, t)


def _sc_hash_kernel(bytes_hbm, idx_hbm, bytes_v, idx_v):
    info = plsc.get_sparse_core_info()
    nc = info.num_cores
    wid = lax.axis_index("s") * nc + lax.axis_index("c")
    nw = nc * info.num_subcores
    rows_per_w = _B // nw

    nbytes = rows_per_w * _LBYTES
    pltpu.sync_copy(bytes_hbm.at[pl.ds(wid * nbytes, nbytes)],
                    bytes_v.at[pl.ds(0, nbytes)])

    def hash_row(rr, _):
        bbase = rr * _LBYTES
        for c in range(13):
            b0 = bytes_v[pl.ds(bbase + c * 16, 16)]
            b1 = bytes_v[pl.ds(bbase + c * 16 + 1, 16)]
            b2 = bytes_v[pl.ds(bbase + c * 16 + 2, 16)]
            idx_v[pl.ds(bbase + c * 16, 16)] = _hash16(b0, b1, b2)
        return 0

    lax.fori_loop(0, rows_per_w, hash_row, 0)
    pltpu.sync_copy(idx_v.at[pl.ds(0, nbytes)],
                    idx_hbm.at[pl.ds(wid * nbytes, nbytes)])


def _sc_gather_kernel(idx_hbm, table_hbm, out_hbm,
                      idx_v, rows0, rows1, sg0, sg1, so0, so1):
    info = plsc.get_sparse_core_info()
    nc = info.num_cores
    wid = lax.axis_index("s") * nc + lax.axis_index("c")
    nw = nc * info.num_subcores
    rows_per_w = _B // nw

    nidx = rows_per_w * _LBYTES
    pltpu.sync_copy(idx_hbm.at[pl.ds(wid * nidx, nidx)],
                    idx_v.at[pl.ds(0, nidx)])

    wbase = wid * rows_per_w

    def drain(rows_v, sem):
        pltpu.make_async_copy(rows_v, out_hbm.at[wbase], sem).wait()

    def gather(rr, rows_v, sem):
        ib = rr * _LBYTES
        c0 = pltpu.async_copy(table_hbm.at[idx_v.at[pl.ds(ib, 128)]],
                              rows_v.at[pl.ds(0, 128)], sem)
        c1 = pltpu.async_copy(table_hbm.at[idx_v.at[pl.ds(ib + 128, 70)]],
                              rows_v.at[pl.ds(128, 70)], sem)
        return c0, c1

    def pair(i, _):
        @pl.when(i > 0)
        def _():
            drain(rows0, so0)
            drain(rows1, so1)
        g0a, g0b = gather(2 * i, rows0, sg0)
        g1a, g1b = gather(2 * i + 1, rows1, sg1)
        g0a.wait()
        g0b.wait()
        pltpu.async_copy(rows0, out_hbm.at[wbase + 2 * i], so0)
        g1a.wait()
        g1b.wait()
        pltpu.async_copy(rows1, out_hbm.at[wbase + 2 * i + 1], so1)
        return 0

    lax.fori_loop(0, rows_per_w // 2, pair, 0)
    drain(rows0, so0)
    drain(rows1, so1)


def kernel(input_bytes, memory_table):
    mesh = plsc.VectorSubcoreMesh(core_axis_name="c", subcore_axis_name="s")
    hash_k = functools.partial(
        pl.kernel,
        mesh=mesh,
        compiler_params=pltpu.CompilerParams(use_tc_tiling_on_sc=False),
        out_type=jax.ShapeDtypeStruct((_B * _LBYTES,), jnp.int32),
        scratch_types=[
            pltpu.VMEM((_B // 32 * _LBYTES + 16,), jnp.int32),
            pltpu.VMEM((_B // 32 * _LBYTES + 16,), jnp.int32),
        ],
    )(_sc_hash_kernel)
    gather_k = functools.partial(
        pl.kernel,
        mesh=mesh,
        compiler_params=pltpu.CompilerParams(use_tc_tiling_on_sc=False),
        out_type=jax.ShapeDtypeStruct((_B, _NOUT, _MEMORY_DIM), jnp.float32),
        scratch_types=[
            pltpu.VMEM((_B // 32 * _LBYTES + 16,), jnp.int32),
            pltpu.VMEM((_NOUT, _MEMORY_DIM), jnp.float32),
            pltpu.VMEM((_NOUT, _MEMORY_DIM), jnp.float32),
            pltpu.SemaphoreType.DMA,
            pltpu.SemaphoreType.DMA,
            pltpu.SemaphoreType.DMA,
            pltpu.SemaphoreType.DMA,
        ],
    )(_sc_gather_kernel)
    idx = hash_k(input_bytes.reshape(-1))
    table_lin = _tc_pack_table(memory_table.T).reshape(_CAPACITY, _MEMORY_DIM)
    return gather_k(idx, table_lin)


_TCOLS = 8192


def _tc_pack_body(x_ref, o_ref):
    y3 = x_ref[...].T.reshape(_TCOLS // 4, 4, _MEMORY_DIM)
    for j in range(4):
        o_ref[:, 32 * j:32 * (j + 1)] = y3[:, j, :]


def _tc_pack_table(table_t):
    return pl.pallas_call(
        _tc_pack_body,
        grid=((_CAPACITY + _TCOLS - 1) // _TCOLS,),
        in_specs=[pl.BlockSpec((_MEMORY_DIM, _TCOLS), lambda i: (0, i))],
        out_specs=pl.BlockSpec((_TCOLS // 4, 128), lambda i: (i, 0)),
        out_shape=jax.ShapeDtypeStruct((_CAPACITY // 4, 128), jnp.float32),
    )(table_t)


# speedup vs baseline: 1.3906x; 1.0132x over previous
import functools

import jax
import jax.numpy as jnp
from jax import lax
from jax.experimental import pallas as pl
from jax.experimental.pallas import tpu as pltpu
from jax.experimental.pallas import tpu_sc as plsc

_CAPACITY = 1000000
_MEMORY_DIM = 32
_B = 4096
_LBYTES = 200
_NOUT = 198


def _hash16(b0, b1, b2):
    t = (1255593 + b0) * 961 + b1 * 31 + b2
    t = t - (t >> 20) * _CAPACITY
    t = t - (t >> 20) * _CAPACITY
    t = t - (t >> 20) * _CAPACITY
    return jnp.where(t >= _CAPACITY, t - _CAPACITY, t)


def _sc_hash_kernel(bytes_hbm, idx_hbm, bytes_v, idx_v):
    info = plsc.get_sparse_core_info()
    nc = info.num_cores
    wid = lax.axis_index("s") * nc + lax.axis_index("c")
    nw = nc * info.num_subcores
    rows_per_w = _B // nw

    nbytes = rows_per_w * _LBYTES
    pltpu.sync_copy(bytes_hbm.at[pl.ds(wid * nbytes, nbytes)],
                    bytes_v.at[pl.ds(0, nbytes)])

    def hash_row(rr, _):
        bbase = rr * _LBYTES
        for c in range(13):
            b0 = bytes_v[pl.ds(bbase + c * 16, 16)]
            b1 = bytes_v[pl.ds(bbase + c * 16 + 1, 16)]
            b2 = bytes_v[pl.ds(bbase + c * 16 + 2, 16)]
            idx_v[pl.ds(bbase + c * 16, 16)] = _hash16(b0, b1, b2)
        return 0

    lax.fori_loop(0, rows_per_w, hash_row, 0)
    pltpu.sync_copy(idx_v.at[pl.ds(0, nbytes)],
                    idx_hbm.at[pl.ds(wid * nbytes, nbytes)])


def _sc_gather_kernel(idx_hbm, table_hbm, out_hbm,
                      idx_v, rows0, rows1, sg0, sg1, so0, so1):
    info = plsc.get_sparse_core_info()
    nc = info.num_cores
    wid = lax.axis_index("s") * nc + lax.axis_index("c")
    nw = nc * info.num_subcores
    rows_per_w = _B // nw

    nidx = rows_per_w * _LBYTES
    pltpu.sync_copy(idx_hbm.at[pl.ds(wid * nidx, nidx)],
                    idx_v.at[pl.ds(0, nidx)])

    wbase = wid * rows_per_w

    def drain(rows_v, sem):
        pltpu.make_async_copy(rows_v, out_hbm.at[wbase], sem).wait()

    def gather(rr, rows_v, sem):
        ib = rr * _LBYTES
        c0 = pltpu.async_copy(table_hbm.at[idx_v.at[pl.ds(ib, 128)]],
                              rows_v.at[pl.ds(0, 128)], sem)
        c1 = pltpu.async_copy(table_hbm.at[idx_v.at[pl.ds(ib + 128, 70)]],
                              rows_v.at[pl.ds(128, 70)], sem)
        return c0, c1

    def pair(i, _):
        @pl.when(i > 0)
        def _():
            drain(rows0, so0)
            drain(rows1, so1)
        g0a, g0b = gather(2 * i, rows0, sg0)
        g1a, g1b = gather(2 * i + 1, rows1, sg1)
        g0a.wait()
        g0b.wait()
        pltpu.async_copy(rows0, out_hbm.at[wbase + 2 * i], so0)
        g1a.wait()
        g1b.wait()
        pltpu.async_copy(rows1, out_hbm.at[wbase + 2 * i + 1], so1)
        return 0

    lax.fori_loop(0, rows_per_w // 2, pair, 0)
    drain(rows0, so0)
    drain(rows1, so1)


def kernel(input_bytes, memory_table):
    mesh = plsc.VectorSubcoreMesh(core_axis_name="c", subcore_axis_name="s")
    hash_k = functools.partial(
        pl.kernel,
        mesh=mesh,
        compiler_params=pltpu.CompilerParams(use_tc_tiling_on_sc=False),
        out_type=jax.ShapeDtypeStruct((_B * _LBYTES,), jnp.int32),
        scratch_types=[
            pltpu.VMEM((_B // 32 * _LBYTES + 16,), jnp.int32),
            pltpu.VMEM((_B // 32 * _LBYTES + 16,), jnp.int32),
        ],
    )(_sc_hash_kernel)
    gather_k = functools.partial(
        pl.kernel,
        mesh=mesh,
        compiler_params=pltpu.CompilerParams(use_tc_tiling_on_sc=False),
        out_type=jax.ShapeDtypeStruct((_B, _NOUT, _MEMORY_DIM), jnp.float32),
        scratch_types=[
            pltpu.VMEM((_B // 32 * _LBYTES + 16,), jnp.int32),
            pltpu.VMEM((_NOUT, _MEMORY_DIM), jnp.float32),
            pltpu.VMEM((_NOUT, _MEMORY_DIM), jnp.float32),
            pltpu.SemaphoreType.DMA,
            pltpu.SemaphoreType.DMA,
            pltpu.SemaphoreType.DMA,
            pltpu.SemaphoreType.DMA,
        ],
    )(_sc_gather_kernel)
    idx = hash_k(input_bytes.reshape(-1))
    table_lin = _tc_pack_table(memory_table.T).reshape(_CAPACITY, _MEMORY_DIM)
    return gather_k(idx, table_lin)


_TCOLS = 16384


def _tc_pack_body(x_ref, o_ref):
    y3 = x_ref[...].T.reshape(_TCOLS // 4, 4, _MEMORY_DIM)
    for j in range(4):
        o_ref[:, 32 * j:32 * (j + 1)] = y3[:, j, :]


def _tc_pack_table(table_t):
    return pl.pallas_call(
        _tc_pack_body,
        grid=((_CAPACITY + _TCOLS - 1) // _TCOLS,),
        in_specs=[pl.BlockSpec((_MEMORY_DIM, _TCOLS), lambda i: (0, i))],
        out_specs=pl.BlockSpec((_TCOLS // 4, 128), lambda i: (i, 0)),
        out_shape=jax.ShapeDtypeStruct((_CAPACITY // 4, 128), jnp.float32),
    )(table_t)


# speedup vs baseline: 1.3972x; 1.0048x over previous
import functools

import jax
import jax.numpy as jnp
from jax import lax
from jax.experimental import pallas as pl
from jax.experimental.pallas import tpu as pltpu
from jax.experimental.pallas import tpu_sc as plsc

_CAPACITY = 1000000
_MEMORY_DIM = 32
_B = 4096
_LBYTES = 200
_NOUT = 198


def _hash16(b0, b1, b2):
    t = (1255593 + b0) * 961 + b1 * 31 + b2
    t = t - (t >> 20) * _CAPACITY
    t = t - (t >> 20) * _CAPACITY
    t = t - (t >> 20) * _CAPACITY
    return jnp.where(t >= _CAPACITY, t - _CAPACITY, t)


def _sc_hash_kernel(bytes_hbm, idx_hbm, bytes_v, idx_v):
    info = plsc.get_sparse_core_info()
    nc = info.num_cores
    wid = lax.axis_index("s") * nc + lax.axis_index("c")
    nw = nc * info.num_subcores
    rows_per_w = _B // nw

    nbytes = rows_per_w * _LBYTES
    pltpu.sync_copy(bytes_hbm.at[pl.ds(wid * nbytes, nbytes)],
                    bytes_v.at[pl.ds(0, nbytes)])

    def hash_row(rr, _):
        bbase = rr * _LBYTES
        for c in range(13):
            b0 = bytes_v[pl.ds(bbase + c * 16, 16)]
            b1 = bytes_v[pl.ds(bbase + c * 16 + 1, 16)]
            b2 = bytes_v[pl.ds(bbase + c * 16 + 2, 16)]
            idx_v[pl.ds(bbase + c * 16, 16)] = _hash16(b0, b1, b2)
        return 0

    lax.fori_loop(0, rows_per_w, hash_row, 0)
    pltpu.sync_copy(idx_v.at[pl.ds(0, nbytes)],
                    idx_hbm.at[pl.ds(wid * nbytes, nbytes)])


def _sc_gather_kernel(idx_hbm, table_hbm, out_hbm,
                      idx_v, rows0, rows1, sg0, sg1, so0, so1):
    info = plsc.get_sparse_core_info()
    nc = info.num_cores
    wid = lax.axis_index("s") * nc + lax.axis_index("c")
    nw = nc * info.num_subcores
    rows_per_w = _B // nw

    nidx = rows_per_w * _LBYTES
    pltpu.sync_copy(idx_hbm.at[pl.ds(wid * nidx, nidx)],
                    idx_v.at[pl.ds(0, nidx)])

    wbase = wid * rows_per_w

    def drain(rows_v, sem):
        pltpu.make_async_copy(rows_v, out_hbm.at[wbase], sem).wait()

    def gather(rr, rows_v, sem):
        ib = rr * _LBYTES
        c0 = pltpu.async_copy(table_hbm.at[idx_v.at[pl.ds(ib, 128)]],
                              rows_v.at[pl.ds(0, 128)], sem)
        c1 = pltpu.async_copy(table_hbm.at[idx_v.at[pl.ds(ib + 128, 70)]],
                              rows_v.at[pl.ds(128, 70)], sem)
        return c0, c1

    def pair(i, _):
        @pl.when(i > 0)
        def _():
            drain(rows0, so0)
            drain(rows1, so1)
        g0a, g0b = gather(2 * i, rows0, sg0)
        g1a, g1b = gather(2 * i + 1, rows1, sg1)
        g0a.wait()
        g0b.wait()
        pltpu.async_copy(rows0, out_hbm.at[wbase + 2 * i], so0)
        g1a.wait()
        g1b.wait()
        pltpu.async_copy(rows1, out_hbm.at[wbase + 2 * i + 1], so1)
        return 0

    lax.fori_loop(0, rows_per_w // 2, pair, 0)
    drain(rows0, so0)
    drain(rows1, so1)


def kernel(input_bytes, memory_table):
    mesh = plsc.VectorSubcoreMesh(core_axis_name="c", subcore_axis_name="s")
    hash_k = functools.partial(
        pl.kernel,
        mesh=mesh,
        compiler_params=pltpu.CompilerParams(use_tc_tiling_on_sc=False),
        out_type=jax.ShapeDtypeStruct((_B * _LBYTES,), jnp.int32),
        scratch_types=[
            pltpu.VMEM((_B // 32 * _LBYTES + 16,), jnp.int32),
            pltpu.VMEM((_B // 32 * _LBYTES + 16,), jnp.int32),
        ],
    )(_sc_hash_kernel)
    gather_k = functools.partial(
        pl.kernel,
        mesh=mesh,
        compiler_params=pltpu.CompilerParams(use_tc_tiling_on_sc=False),
        out_type=jax.ShapeDtypeStruct((_B, _NOUT, _MEMORY_DIM), jnp.float32),
        scratch_types=[
            pltpu.VMEM((_B // 32 * _LBYTES + 16,), jnp.int32),
            pltpu.VMEM((_NOUT, _MEMORY_DIM), jnp.float32),
            pltpu.VMEM((_NOUT, _MEMORY_DIM), jnp.float32),
            pltpu.SemaphoreType.DMA,
            pltpu.SemaphoreType.DMA,
            pltpu.SemaphoreType.DMA,
            pltpu.SemaphoreType.DMA,
        ],
    )(_sc_gather_kernel)
    idx = hash_k(input_bytes.reshape(-1))
    table_lin = _tc_pack_table(memory_table.T).reshape(_CAPACITY, _MEMORY_DIM)
    return gather_k(idx, table_lin)


_TCOLS = 24576


def _tc_pack_body(x_ref, o_ref):
    y3 = x_ref[...].T.reshape(_TCOLS // 4, 4, _MEMORY_DIM)
    for j in range(4):
        o_ref[:, 32 * j:32 * (j + 1)] = y3[:, j, :]


def _tc_pack_table(table_t):
    return pl.pallas_call(
        _tc_pack_body,
        grid=((_CAPACITY + _TCOLS - 1) // _TCOLS,),
        in_specs=[pl.BlockSpec((_MEMORY_DIM, _TCOLS), lambda i: (0, i))],
        out_specs=pl.BlockSpec((_TCOLS // 4, 128), lambda i: (i, 0)),
        out_shape=jax.ShapeDtypeStruct((_CAPACITY // 4, 128), jnp.float32),
    )(table_t)
